# Initial kernel scaffold; baseline (speedup 1.0000x reference)
import jax
import jax.numpy as jnp
from jax.experimental import pallas as pl


def kernel(output, target):
    raise NotImplementedError("write your pallas kernel here")


# speedup vs baseline: 4.4719x; 4.4719x over previous
import functools

import jax
import jax.numpy as jnp
from jax.experimental import pallas as pl
from jax.experimental.pallas import tpu as pltpu

_EPS = 0.1
_V = 32000
_N = 2048
_IGNORE = 0
_CONF = 1.0 - _EPS
_EPS_U = _EPS / (_V - 1)

_BR = 128
_BV = 3200
_NR = _N // _BR
_NV = _V // _BV


def _loss_kernel(x_ref, tgt_ref, out_ref, m_ref, s_ref, t_ref, g_ref):
    i = pl.program_id(0)
    j = pl.program_id(1)

    x = x_ref[...]

    @pl.when(j == 0)
    def _init():
        m_ref[...] = jnp.full((_BR, 1), -jnp.inf, jnp.float32)
        s_ref[...] = jnp.zeros((_BR, 1), jnp.float32)
        t_ref[...] = jnp.zeros((_BR, 1), jnp.float32)
        g_ref[...] = jnp.zeros((_BR, 1), jnp.float32)

    m_old = m_ref[...]
    blk_max = jnp.max(x, axis=1, keepdims=True)
    m_new = jnp.maximum(m_old, blk_max)
    s_ref[...] = (s_ref[...] * jnp.exp(m_old - m_new)
                  + jnp.sum(jnp.exp(x - m_new), axis=1, keepdims=True))
    m_ref[...] = m_new
    t_ref[...] = t_ref[...] + jnp.sum(x, axis=1, keepdims=True)

    tgt = tgt_ref[i]
    cols = jax.lax.broadcasted_iota(jnp.int32, (_BR, _BV), 1) + j * _BV
    g_ref[...] = g_ref[...] + jnp.sum(
        jnp.where(cols == tgt, x, 0.0), axis=1, keepdims=True)

    @pl.when(j == _NV - 1)
    def _finish():
        lse = m_ref[...] + jnp.log(s_ref[...])
        gp = g_ref[...] - lse
        loss_rows = (_EPS_U * (_V * lse - t_ref[...])
                     - (_CONF - _EPS_U) * gp)
        loss_rows = jnp.where(tgt == _IGNORE, 0.0, loss_rows)
        part = jnp.sum(loss_rows) * (1.0 / _N)

        @pl.when(i == 0)
        def _first():
            out_ref[0, 0] = part

        @pl.when(i > 0)
        def _rest():
            out_ref[0, 0] = out_ref[0, 0] + part


@functools.partial(jax.jit, static_argnames=())
def kernel(output, target):
    tgt3 = target.reshape(_NR, _BR, 1)
    out = pl.pallas_call(
        _loss_kernel,
        grid=(_NR, _NV),
        in_specs=[
            pl.BlockSpec((_BR, _BV), lambda i, j: (i, j)),
            pl.BlockSpec((_NR, _BR, 1), lambda i, j: (0, 0, 0)),
        ],
        out_specs=pl.BlockSpec((1, 1), lambda i, j: (0, 0),
                               memory_space=pltpu.SMEM),
        out_shape=jax.ShapeDtypeStruct((1, 1), jnp.float32),
        scratch_shapes=[
            pltpu.VMEM((_BR, 1), jnp.float32),
            pltpu.VMEM((_BR, 1), jnp.float32),
            pltpu.VMEM((_BR, 1), jnp.float32),
            pltpu.VMEM((_BR, 1), jnp.float32),
        ],
    )(output, tgt3)
    return out[0, 0]
